# baseline (speedup 1.0000x reference)
from functools import partial

import jax
import jax.numpy as jnp
from jax.experimental import pallas as pl
from jax.experimental.pallas import tpu as pltpu


def _body(tT_ref, tTs_ref, n1_ref, pT_ref, rT_ref, c_ref, w1bT_ref, w2b_ref,
          out_ref, s1_ref, m_ref, *, nsteps, rem, inv_temp):
    k = pl.program_id(0)

    @pl.when(k == 0)
    def _init():
        s1_ref[...] = jnp.zeros_like(s1_ref)
        m_ref[...] = jnp.zeros_like(m_ref)

    tT = tT_ref[...]
    tTs = tTs_ref[...]
    kt = rT_ref.shape[0]

    def _accum(n1, rT):
        q = jax.lax.dot_general(
            n1[:-1, :], tTs, (((0,), (0,)), ((), ())),
            preferred_element_type=jnp.float32)
        s1_ref[...] += jnp.sum(jnp.exp2(q), axis=0, keepdims=True)
        m_ref[...] += jax.lax.dot_general(
            n1, rT.astype(jnp.bfloat16), (((1,), (0,)), ((), ())),
            preferred_element_type=jnp.float32)

    @pl.when(k < nsteps - 1)
    def _full_tile():
        _accum(n1_ref[:, pl.ds(k * kt, kt)], rT_ref[...])

    @pl.when(k == nsteps - 1)
    def _tail_and_epilogue():
        _accum(n1_ref[:, pl.ds((nsteps - 1) * kt, rem)],
               rT_ref[pl.ds(0, rem), :])

        s1 = s1_ref[...]
        m = m_ref[...]
        nd = tT.shape[0]
        s3 = m[nd:, :]
        s2 = jnp.sum(tT * m[:nd, :], axis=0, keepdims=True) * inv_temp
        loss = (s2 - jnp.log(s1) * s3) / (s3 + 1e-8)

        c = c_ref[...]

        def _soft(xT):
            gT = jax.lax.dot_general(
                c, xT, (((1,), (0,)), ((), ())),
                preferred_element_type=jnp.float32)
            gT = jnp.exp(gT - jnp.max(gT, axis=0, keepdims=True))
            return gT / jnp.sum(gT, axis=0, keepdims=True)

        svT = (_soft(tT) - _soft(pT_ref[...])) ** 2
        nb = svT.shape[1]
        ones = jnp.ones((1, nb), jnp.float32)
        hT = jax.lax.dot_general(
            w1bT_ref[...], jnp.concatenate([svT, ones], axis=0),
            (((0,), (0,)), ((), ())),
            preferred_element_type=jnp.float32)
        hT = jnp.maximum(hT, 0.0)
        zT = jax.lax.dot_general(
            w2b_ref[...], jnp.concatenate([hT, ones], axis=0),
            (((1,), (0,)), ((), ())),
            preferred_element_type=jnp.float32)
        piw = jax.nn.softplus(zT)
        piw_sum = jnp.sum(piw, axis=1, keepdims=True)
        num = jnp.sum(loss * piw, axis=1, keepdims=True)
        out_ref[...] = -num / (piw_sum + 1e-8)


def kernel(target_emb, neighbor_emb, present_user_emb, rating_mat,
           cluster, W1, b1, W2, b2):
    B, D = target_emb.shape
    K = neighbor_emb.shape[0]
    C = cluster.shape[0]
    KT = 2048
    nsteps = pl.cdiv(K, KT)
    rem = K - (nsteps - 1) * KT

    out = pl.pallas_call(
        partial(_body, nsteps=nsteps, rem=rem, inv_temp=1.0 / 5.0),
        grid=(nsteps,),
        in_specs=[
            pl.BlockSpec((D, B), lambda k: (0, 0)),
            pl.BlockSpec((D, B), lambda k: (0, 0)),
            pl.BlockSpec((D + 1, K), lambda k: (0, 0)),
            pl.BlockSpec((D, B), lambda k: (0, 0)),
            pl.BlockSpec((KT, B), lambda k: (k, 0)),
            pl.BlockSpec((C, D), lambda k: (0, 0)),
            pl.BlockSpec((C + 1, D), lambda k: (0, 0)),
            pl.BlockSpec((1, D + 1), lambda k: (0, 0)),
        ],
        out_specs=pl.BlockSpec((1, 1), lambda k: (0, 0)),
        out_shape=jax.ShapeDtypeStruct((1, 1), jnp.float32),
        scratch_shapes=[
            pltpu.VMEM((1, B), jnp.float32),
            pltpu.VMEM((D + 1, B), jnp.float32),
        ],
        compiler_params=pltpu.CompilerParams(
            dimension_semantics=("arbitrary",)),
    )(target_emb.T,
      (target_emb.T * (1.0 / 5.0 * 1.4426950408889634)).astype(jnp.bfloat16),
      jnp.concatenate(
          [neighbor_emb.T, jnp.ones((1, K), jnp.float32)],
          axis=0).astype(jnp.bfloat16),
      present_user_emb.T, rating_mat.T,
      cluster,
      jnp.concatenate([W1.T, b1[None, :]], axis=0),
      jnp.concatenate([W2, b2[:, None]], axis=1))
    return out[0, 0]


# speedup vs baseline: 1.1465x; 1.1465x over previous
from functools import partial

import jax
import jax.numpy as jnp
from jax.experimental import pallas as pl
from jax.experimental.pallas import tpu as pltpu


def _body(tT_ref, tTs_ref, nTa_ref, nTb_ref, pT_ref, rTa_ref, rTb_ref,
          c_ref, w1bT_ref, w2b_ref, out_ref, s1_ref, m_ref,
          *, nsteps, rem, inv_temp):
    k = pl.program_id(0)

    @pl.when(k == 0)
    def _init():
        s1_ref[...] = jnp.zeros_like(s1_ref)
        m_ref[...] = jnp.zeros_like(m_ref)

    tT = tT_ref[...]
    tTs = tTs_ref[...]

    def _accum(nT, rT):
        nTb16 = nT.astype(jnp.bfloat16)
        q = jax.lax.dot_general(
            nTb16, tTs, (((0,), (0,)), ((), ())),
            preferred_element_type=jnp.float32)
        s1_ref[...] += jnp.sum(jnp.exp2(q), axis=0, keepdims=True)
        n1 = jnp.concatenate(
            [nTb16, jnp.ones((1, nT.shape[1]), jnp.bfloat16)], axis=0)
        m_ref[...] += jax.lax.dot_general(
            n1, rT.astype(jnp.bfloat16), (((1,), (0,)), ((), ())),
            preferred_element_type=jnp.float32)

    _accum(nTa_ref[...], rTa_ref[...])

    @pl.when(k < nsteps - 1)
    def _b_full():
        _accum(nTb_ref[...], rTb_ref[...])

    @pl.when(k == nsteps - 1)
    def _b_tail_and_epilogue():
        _accum(nTb_ref[:, :rem], rTb_ref[:rem, :])

        s1 = s1_ref[...]
        m = m_ref[...]
        nd = tT.shape[0]
        s3 = m[nd:, :]
        s2 = jnp.sum(tT * m[:nd, :], axis=0, keepdims=True) * inv_temp
        loss = (s2 - jnp.log(s1) * s3) / (s3 + 1e-8)

        c = c_ref[...]

        def _soft(xT):
            gT = jax.lax.dot_general(
                c, xT, (((1,), (0,)), ((), ())),
                preferred_element_type=jnp.float32)
            gT = jnp.exp(gT - jnp.max(gT, axis=0, keepdims=True))
            return gT / jnp.sum(gT, axis=0, keepdims=True)

        svT = (_soft(tT) - _soft(pT_ref[...])) ** 2
        nb = svT.shape[1]
        ones = jnp.ones((1, nb), jnp.float32)
        hT = jax.lax.dot_general(
            w1bT_ref[...], jnp.concatenate([svT, ones], axis=0),
            (((0,), (0,)), ((), ())),
            preferred_element_type=jnp.float32)
        hT = jnp.maximum(hT, 0.0)
        zT = jax.lax.dot_general(
            w2b_ref[...], jnp.concatenate([hT, ones], axis=0),
            (((1,), (0,)), ((), ())),
            preferred_element_type=jnp.float32)
        piw = jax.nn.softplus(zT)
        piw_sum = jnp.sum(piw, axis=1, keepdims=True)
        num = jnp.sum(loss * piw, axis=1, keepdims=True)
        out_ref[...] = -num / (piw_sum + 1e-8)


def kernel(target_emb, neighbor_emb, present_user_emb, rating_mat,
           cluster, W1, b1, W2, b2):
    B, D = target_emb.shape
    K = neighbor_emb.shape[0]
    C = cluster.shape[0]
    KT = 1536
    nsteps = pl.cdiv(pl.cdiv(K, KT), 2)
    rem = K - (2 * nsteps - 1) * KT

    ns = nsteps
    out = pl.pallas_call(
        partial(_body, nsteps=nsteps, rem=rem, inv_temp=1.0 / 5.0),
        grid=(nsteps,),
        in_specs=[
            pl.BlockSpec((D, B), lambda k: (0, 0)),
            pl.BlockSpec((D, B), lambda k: (0, 0)),
            pl.BlockSpec((D, KT), lambda k: (0, k)),
            pl.BlockSpec((D, KT), lambda k: (0, k + ns)),
            pl.BlockSpec((D, B), lambda k: (0, 0)),
            pl.BlockSpec((KT, B), lambda k: (k, 0)),
            pl.BlockSpec((KT, B), lambda k: (k + ns, 0)),
            pl.BlockSpec((C, D), lambda k: (0, 0)),
            pl.BlockSpec((C + 1, D), lambda k: (0, 0)),
            pl.BlockSpec((1, D + 1), lambda k: (0, 0)),
        ],
        out_specs=pl.BlockSpec((1, 1), lambda k: (0, 0)),
        out_shape=jax.ShapeDtypeStruct((1, 1), jnp.float32),
        scratch_shapes=[
            pltpu.VMEM((1, B), jnp.float32),
            pltpu.VMEM((D + 1, B), jnp.float32),
        ],
        compiler_params=pltpu.CompilerParams(
            dimension_semantics=("arbitrary",)),
    )(target_emb.T,
      (target_emb.T * (1.0 / 5.0 * 1.4426950408889634)).astype(jnp.bfloat16),
      neighbor_emb.T, neighbor_emb.T,
      present_user_emb.T, rating_mat.T, rating_mat.T,
      cluster,
      jnp.concatenate([W1.T, b1[None, :]], axis=0),
      jnp.concatenate([W2, b2[:, None]], axis=1))
    return out[0, 0]


# speedup vs baseline: 1.1980x; 1.0450x over previous
from functools import partial

import jax
import jax.numpy as jnp
from jax.experimental import pallas as pl
from jax.experimental.pallas import tpu as pltpu


def _body(tT_ref, tTs_ref, nTa_ref, pT_ref, rTa_ref,
          c_ref, w1bT_ref, w2b_ref, out_ref, s1_ref, m_ref,
          *, nsteps, rem, inv_temp):
    k = pl.program_id(0)

    @pl.when(k == 0)
    def _init():
        s1_ref[...] = jnp.zeros_like(s1_ref)
        m_ref[...] = jnp.zeros_like(m_ref)

    tT = tT_ref[...]
    tTs = tTs_ref[...]

    def _accum(nT, rT):
        nTb16 = nT.astype(jnp.bfloat16)
        q = jax.lax.dot_general(
            nTb16, tTs, (((0,), (0,)), ((), ())),
            preferred_element_type=jnp.float32)
        s1_ref[...] += jnp.sum(jnp.exp2(q), axis=0, keepdims=True)
        n1 = jnp.concatenate(
            [nT, jnp.ones((1, nT.shape[1]), jnp.float32)], axis=0)
        m_ref[...] += jax.lax.dot_general(
            n1, rT, (((1,), (0,)), ((), ())),
            preferred_element_type=jnp.float32)

    @pl.when(k < nsteps - 1)
    def _full_tile():
        _accum(nTa_ref[...], rTa_ref[...])

    @pl.when(k == nsteps - 1)
    def _tail_and_epilogue():
        _accum(nTa_ref[:, :rem], rTa_ref[:rem, :])

        s1 = s1_ref[...]
        m = m_ref[...]
        nd = tT.shape[0]
        s3 = m[nd:, :]
        s2 = jnp.sum(tT * m[:nd, :], axis=0, keepdims=True) * inv_temp
        loss = (s2 - jnp.log(s1) * s3) / (s3 + 1e-8)

        c = c_ref[...]

        def _soft(xT):
            gT = jax.lax.dot_general(
                c, xT, (((1,), (0,)), ((), ())),
                preferred_element_type=jnp.float32)
            gT = jnp.exp(gT - jnp.max(gT, axis=0, keepdims=True))
            return gT / jnp.sum(gT, axis=0, keepdims=True)

        svT = (_soft(tT) - _soft(pT_ref[...])) ** 2
        nb = svT.shape[1]
        ones = jnp.ones((1, nb), jnp.float32)
        hT = jax.lax.dot_general(
            w1bT_ref[...], jnp.concatenate([svT, ones], axis=0),
            (((0,), (0,)), ((), ())),
            preferred_element_type=jnp.float32)
        hT = jnp.maximum(hT, 0.0)
        zT = jax.lax.dot_general(
            w2b_ref[...], jnp.concatenate([hT, ones], axis=0),
            (((1,), (0,)), ((), ())),
            preferred_element_type=jnp.float32)
        piw = jax.nn.softplus(zT)
        piw_sum = jnp.sum(piw, axis=1, keepdims=True)
        num = jnp.sum(loss * piw, axis=1, keepdims=True)
        out_ref[...] = -num / (piw_sum + 1e-8)


def kernel(target_emb, neighbor_emb, present_user_emb, rating_mat,
           cluster, W1, b1, W2, b2):
    B, D = target_emb.shape
    K = neighbor_emb.shape[0]
    C = cluster.shape[0]
    KT = 4096
    nsteps = pl.cdiv(K, KT)
    rem = K - (nsteps - 1) * KT

    out = pl.pallas_call(
        partial(_body, nsteps=nsteps, rem=rem, inv_temp=1.0 / 5.0),
        grid=(nsteps,),
        in_specs=[
            pl.BlockSpec((D, B), lambda k: (0, 0)),
            pl.BlockSpec((D, B), lambda k: (0, 0)),
            pl.BlockSpec((D, KT), lambda k: (0, k)),
            pl.BlockSpec((D, B), lambda k: (0, 0)),
            pl.BlockSpec((KT, B), lambda k: (k, 0)),
            pl.BlockSpec((C, D), lambda k: (0, 0)),
            pl.BlockSpec((C + 1, D), lambda k: (0, 0)),
            pl.BlockSpec((1, D + 1), lambda k: (0, 0)),
        ],
        out_specs=pl.BlockSpec((1, 1), lambda k: (0, 0)),
        out_shape=jax.ShapeDtypeStruct((1, 1), jnp.float32),
        scratch_shapes=[
            pltpu.VMEM((1, B), jnp.float32),
            pltpu.VMEM((D + 1, B), jnp.float32),
        ],
        compiler_params=pltpu.CompilerParams(
            dimension_semantics=("arbitrary",)),
    )(target_emb.T,
      (target_emb.T * (1.0 / 5.0 * 1.4426950408889634)).astype(jnp.bfloat16),
      neighbor_emb.T,
      present_user_emb.T, rating_mat.T,
      cluster,
      jnp.concatenate([W1.T, b1[None, :]], axis=0),
      jnp.concatenate([W2, b2[:, None]], axis=1))
    return out[0, 0]
